# baseline (speedup 1.0000x reference)
import functools

import jax
import jax.numpy as jnp
from jax import lax
from jax.experimental import pallas as pl
from jax.experimental.pallas import tpu as pltpu
from jax.experimental.pallas import tpu_sc as plsc

LANES = 16


def _build_sc_kernel(B, L, V, D):
    info = plsc.get_sparse_core_info()
    NC, NS = info.num_cores, info.num_subcores
    NW = NC * NS
    BL = B * L
    assert B % NW == 0
    rows_per_w = B // NW
    toks_per_w = rows_per_w * L
    CHUNK = 128
    assert toks_per_w % CHUNK == 0
    n_chunks = toks_per_w // CHUNK
    n_groups = toks_per_w // LANES
    assert D % LANES == 0
    KD = D // LANES
    GPC = CHUNK // LANES

    mesh = plsc.VectorSubcoreMesh(core_axis_name="c", subcore_axis_name="s")

    @functools.partial(
        pl.kernel,
        mesh=mesh,
        compiler_params=pltpu.CompilerParams(use_tc_tiling_on_sc=False),
        out_type=jax.ShapeDtypeStruct((BL, D), jnp.float32),
        scratch_types=[
            pltpu.VMEM((rows_per_w + LANES,), jnp.int32),
            pltpu.VMEM((L, D), jnp.float32),
            pltpu.VMEM((toks_per_w,), jnp.float32),
            pltpu.VMEM((CHUNK,), jnp.int32),
            pltpu.VMEM((CHUNK, D), jnp.float32),
            pltpu.SemaphoreType.DMA,
        ],
    )
    def k(x_hbm, lens_hbm, emb_hbm, pe_hbm, out_hbm,
          lens_v, pe_v, mf_v, idx_v, rows_v, sem):
        wid = lax.axis_index("s") * NC + lax.axis_index("c")
        base_row = wid * rows_per_w
        base_tok = wid * toks_per_w

        pltpu.sync_copy(lens_hbm.at[pl.ds(base_row, rows_per_w)],
                        lens_v.at[pl.ds(0, rows_per_w)])
        pltpu.sync_copy(pe_hbm, pe_v)

        iota = lax.iota(jnp.int32, LANES)
        one = jnp.int32(1)
        zero = jnp.int32(0)

        def mask_group(g, carry):
            r0, l0 = carry
            l_raw = l0 + iota
            wrap = l_raw >= L
            l = jnp.where(wrap, l_raw - L, l_raw)
            rsel = jnp.where(wrap, one, zero)
            lens16 = lens_v[pl.ds(r0, LANES)]
            ln = lens16.at[rsel].get(mode="promise_in_bounds")
            mf = jnp.where(l < ln, jnp.float32(8.0), jnp.float32(0.0))
            mf_v[pl.ds(g * LANES, LANES)] = mf
            l0n = l0 + LANES
            over = l0n >= L
            l0n = jnp.where(over, l0n - L, l0n)
            r0n = jnp.where(over, r0 + one, r0)
            return (r0n, l0n)

        lax.fori_loop(0, n_groups, mask_group, (zero, zero))

        def chunk(c, lb):
            off = base_tok + c * CHUNK
            coff = c * CHUNK
            pltpu.sync_copy(x_hbm.at[pl.ds(off, CHUNK)], idx_v)
            pltpu.async_copy(emb_hbm.at[idx_v], rows_v, sem).wait()

            def group(g, lbg):
                gbase = g * LANES
                mf16 = mf_v[pl.ds(coff + gbase, LANES)]
                for jj in range(LANES):
                    j = gbase + jj
                    sel = jnp.full((LANES,), jj, dtype=jnp.int32)
                    mf = mf16.at[sel].get(mode="promise_in_bounds")
                    mp = mf * jnp.float32(0.125)
                    l_raw = lbg + jj
                    l = jnp.where(l_raw >= L, l_raw - L, l_raw)
                    for kk in range(KD):
                        s = pl.ds(kk * LANES, LANES)
                        rows_v[j, s] = rows_v[j, s] * mf + pe_v[l, s] * mp
                lbn = lbg + LANES
                return jnp.where(lbn >= L, lbn - L, lbn)

            lax.fori_loop(0, GPC, group, lb)
            pltpu.sync_copy(rows_v, out_hbm.at[pl.ds(off, CHUNK)])
            lbn = lb + CHUNK
            return jnp.where(lbn >= L, lbn - L, lbn)

        lax.fori_loop(0, n_chunks, chunk, zero)

    return k


def kernel(x, input_lengths, embedding_weight, pos_enc):
    B, L = x.shape
    V, D = embedding_weight.shape
    k = _build_sc_kernel(B, L, V, D)
    out = k(x.reshape(-1), input_lengths, embedding_weight, pos_enc)
    return out.reshape(B, L, D)


# speedup vs baseline: 1.1807x; 1.1807x over previous
import functools

import jax
import jax.numpy as jnp
from jax import lax
from jax.experimental import pallas as pl
from jax.experimental.pallas import tpu as pltpu
from jax.experimental.pallas import tpu_sc as plsc

LANES = 16


def _build_sc_kernel(B, L, V, D):
    info = plsc.get_sparse_core_info()
    NC, NS = info.num_cores, info.num_subcores
    NW = NC * NS
    BL = B * L
    assert B % NW == 0
    rows_per_w = B // NW
    toks_per_w = rows_per_w * L
    CHUNK = 128
    assert toks_per_w % (2 * CHUNK) == 0
    n_chunks = toks_per_w // CHUNK
    n_steps = n_chunks // 2
    n_groups = toks_per_w // LANES
    assert D % LANES == 0
    KD = D // LANES
    GPC = CHUNK // LANES

    mesh = plsc.VectorSubcoreMesh(core_axis_name="c", subcore_axis_name="s")

    @functools.partial(
        pl.kernel,
        mesh=mesh,
        compiler_params=pltpu.CompilerParams(use_tc_tiling_on_sc=False),
        out_type=jax.ShapeDtypeStruct((BL, D), jnp.float32),
        scratch_types=[
            pltpu.VMEM((rows_per_w + LANES,), jnp.int32),
            pltpu.VMEM((L, D), jnp.float32),
            pltpu.VMEM((toks_per_w,), jnp.float32),
            pltpu.VMEM((CHUNK,), jnp.int32),
            pltpu.VMEM((CHUNK,), jnp.int32),
            pltpu.VMEM((CHUNK, D), jnp.float32),
            pltpu.VMEM((CHUNK, D), jnp.float32),
            pltpu.SemaphoreType.DMA,
            pltpu.SemaphoreType.DMA,
            pltpu.SemaphoreType.DMA,
            pltpu.SemaphoreType.DMA,
            pltpu.SemaphoreType.DMA,
            pltpu.SemaphoreType.DMA,
        ],
    )
    def k(x_hbm, lens_hbm, emb_hbm, pe_hbm, out_hbm,
          lens_v, pem_v, mf_v, idx0, idx1, rows0, rows1,
          sem_g0, sem_g1, sem_s0, sem_s1, sem_i0, sem_i1):
        idx = (idx0, idx1)
        rows = (rows0, rows1)
        sem_g = (sem_g0, sem_g1)
        sem_s = (sem_s0, sem_s1)
        sem_i = (sem_i0, sem_i1)

        wid = lax.axis_index("s") * NC + lax.axis_index("c")
        base_row = wid * rows_per_w
        base_tok = wid * toks_per_w

        pltpu.sync_copy(lens_hbm.at[pl.ds(base_row, rows_per_w)],
                        lens_v.at[pl.ds(0, rows_per_w)])
        pltpu.sync_copy(pe_hbm, pem_v)

        iota = lax.iota(jnp.int32, LANES)
        one = jnp.int32(1)
        zero = jnp.int32(0)

        def pe_scale(i, carry):
            row = i
            for kk in range(KD):
                s = pl.ds(kk * LANES, LANES)
                pem_v[row, s] = pem_v[row, s] * jnp.float32(0.125)
            return carry

        lax.fori_loop(0, L, pe_scale, zero)

        def mask_group(g, carry):
            r0, l0 = carry
            l_raw = l0 + iota
            wrap = l_raw >= L
            l = jnp.where(wrap, l_raw - L, l_raw)
            rsel = jnp.where(wrap, one, zero)
            lens16 = lens_v[pl.ds(r0, LANES)]
            ln = lens16.at[rsel].get(mode="promise_in_bounds")
            mf = jnp.where(l < ln, jnp.float32(8.0), jnp.float32(0.0))
            mf_v[pl.ds(g * LANES, LANES)] = mf
            l0n = l0 + LANES
            over = l0n >= L
            l0n = jnp.where(over, l0n - L, l0n)
            r0n = jnp.where(over, r0 + one, r0)
            return (r0n, l0n)

        lax.fori_loop(0, n_groups, mask_group, (zero, zero))

        def gather_start(c, b):
            pltpu.async_copy(emb_hbm.at[idx[b]], rows[b], sem_g[b])

        def gather_wait(c, b):
            pltpu.make_async_copy(emb_hbm.at[idx[b]], rows[b], sem_g[b]).wait()

        def idx_start(c, b):
            pltpu.async_copy(x_hbm.at[pl.ds(base_tok + c * CHUNK, CHUNK)],
                             idx[b], sem_i[b])

        def idx_wait(c, b):
            pltpu.make_async_copy(x_hbm.at[pl.ds(base_tok + c * CHUNK, CHUNK)],
                                  idx[b], sem_i[b]).wait()

        def store_start(c, b):
            pltpu.async_copy(rows[b], out_hbm.at[pl.ds(base_tok + c * CHUNK, CHUNK)],
                             sem_s[b])

        def store_wait(c, b):
            pltpu.make_async_copy(rows[b], out_hbm.at[pl.ds(base_tok + c * CHUNK, CHUNK)],
                                  sem_s[b]).wait()

        def compute(b, coff, lb):
            def group(g, lbg):
                gbase = g * LANES
                mf16 = mf_v[pl.ds(coff + gbase, LANES)]
                for jj in range(LANES):
                    j = gbase + jj
                    sel = jnp.full((LANES,), jj, dtype=jnp.int32)
                    mf = mf16.at[sel].get(mode="promise_in_bounds")
                    l_raw = lbg + jj
                    l = jnp.where(l_raw >= L, l_raw - L, l_raw)
                    for kk in range(KD):
                        s = pl.ds(kk * LANES, LANES)
                        rows[b][j, s] = (rows[b][j, s] + pem_v[l, s]) * mf
                lbn = lbg + LANES
                return jnp.where(lbn >= L, lbn - L, lbn)

            return lax.fori_loop(0, GPC, group, lb)

        pltpu.sync_copy(x_hbm.at[pl.ds(base_tok, CHUNK)], idx0)
        gather_start(zero, 0)
        idx_start(one, 1)

        def step(s, lb):
            lbs = lb
            for cc in range(2):
                b, b1 = cc, 1 - cc
                c = 2 * s + cc
                gather_wait(c, b)
                @pl.when(s < n_steps - 1)
                def _():
                    idx_start(c + 2, b)
                @pl.when((c + 1 < n_chunks) & (c >= 1))
                def _():
                    store_wait(c - 1, b1)
                @pl.when(c + 1 < n_chunks)
                def _():
                    idx_wait(c + 1, b1)
                    gather_start(c + 1, b1)
                lbs = compute(b, c * CHUNK, lbs)
                store_start(c, b)
            return lbs

        lax.fori_loop(0, n_steps, step, zero)

        store_wait(n_chunks - 2, 0)
        store_wait(n_chunks - 1, 1)

    return k


def kernel(x, input_lengths, embedding_weight, pos_enc):
    B, L = x.shape
    V, D = embedding_weight.shape
    k = _build_sc_kernel(B, L, V, D)
    out = k(x.reshape(-1), input_lengths, embedding_weight, pos_enc)
    return out.reshape(B, L, D)


# speedup vs baseline: 1.3960x; 1.1824x over previous
import functools

import jax
import jax.numpy as jnp
from jax import lax
from jax.experimental import pallas as pl
from jax.experimental.pallas import tpu as pltpu
from jax.experimental.pallas import tpu_sc as plsc

LANES = 16


def _build_sc_kernel(B, L, V, D):
    info = plsc.get_sparse_core_info()
    NC, NS = info.num_cores, info.num_subcores
    NW = NC * NS
    BL = B * L
    assert B % NW == 0
    rows_per_w = B // NW
    toks_per_w = rows_per_w * L
    CHUNK = 128
    assert toks_per_w % (2 * CHUNK) == 0
    n_chunks = toks_per_w // CHUNK
    n_steps = n_chunks // 2
    n_groups = toks_per_w // LANES
    assert D % LANES == 0
    KD = D // LANES
    GPC = CHUNK // LANES

    mesh = plsc.VectorSubcoreMesh(core_axis_name="c", subcore_axis_name="s")

    @functools.partial(
        pl.kernel,
        mesh=mesh,
        compiler_params=pltpu.CompilerParams(use_tc_tiling_on_sc=False),
        out_type=jax.ShapeDtypeStruct((BL, D), jnp.float32),
        scratch_types=[
            pltpu.VMEM((rows_per_w + LANES,), jnp.int32),
            pltpu.VMEM((L, D), jnp.float32),
            pltpu.VMEM((toks_per_w,), jnp.float32),
            pltpu.VMEM((CHUNK,), jnp.int32),
            pltpu.VMEM((CHUNK,), jnp.int32),
            pltpu.VMEM((CHUNK, D), jnp.float32),
            pltpu.VMEM((CHUNK, D), jnp.float32),
            pltpu.SemaphoreType.DMA,
            pltpu.SemaphoreType.DMA,
            pltpu.SemaphoreType.DMA,
            pltpu.SemaphoreType.DMA,
            pltpu.SemaphoreType.DMA,
            pltpu.SemaphoreType.DMA,
        ],
    )
    def k(x_hbm, lens_hbm, emb_hbm, pe_hbm, out_hbm,
          lens_v, pem_v, mf_v, idx0, idx1, rows0, rows1,
          sem_g0, sem_g1, sem_s0, sem_s1, sem_i0, sem_i1):
        idx = (idx0, idx1)
        rows = (rows0, rows1)
        sem_g = (sem_g0, sem_g1)
        sem_s = (sem_s0, sem_s1)
        sem_i = (sem_i0, sem_i1)

        wid = lax.axis_index("s") * NC + lax.axis_index("c")
        base_row = wid * rows_per_w
        base_tok = wid * toks_per_w

        pltpu.sync_copy(lens_hbm.at[pl.ds(base_row, rows_per_w)],
                        lens_v.at[pl.ds(0, rows_per_w)])
        pltpu.sync_copy(pe_hbm, pem_v)

        iota = lax.iota(jnp.int32, LANES)
        one = jnp.int32(1)
        zero = jnp.int32(0)

        def pe_scale(i, carry):
            row = i
            for kk in range(KD):
                s = pl.ds(kk * LANES, LANES)
                pem_v[row, s] = pem_v[row, s] * jnp.float32(0.125)
            return carry

        lax.fori_loop(0, L, pe_scale, zero)

        def mask_group(g, carry):
            r0, l0 = carry
            l_raw = l0 + iota
            wrap = l_raw >= L
            l = jnp.where(wrap, l_raw - L, l_raw)
            rsel = jnp.where(wrap, one, zero)
            lens16 = lens_v[pl.ds(r0, LANES)]
            ln = lens16.at[rsel].get(mode="promise_in_bounds")
            mf = jnp.where(l < ln, jnp.float32(8.0), jnp.float32(0.0))
            mf_v[pl.ds(g * LANES, LANES)] = mf
            l0n = l0 + LANES
            over = l0n >= L
            l0n = jnp.where(over, l0n - L, l0n)
            r0n = jnp.where(over, r0 + one, r0)
            return (r0n, l0n)

        lax.fori_loop(0, n_groups, mask_group, (zero, zero))

        def gather_start(c, b):
            pltpu.async_copy(emb_hbm.at[idx[b]], rows[b], sem_g[b])

        def gather_wait(c, b):
            pltpu.make_async_copy(emb_hbm.at[idx[b]], rows[b], sem_g[b]).wait()

        def idx_start(c, b):
            pltpu.async_copy(x_hbm.at[pl.ds(base_tok + c * CHUNK, CHUNK)],
                             idx[b], sem_i[b])

        def idx_wait(c, b):
            pltpu.make_async_copy(x_hbm.at[pl.ds(base_tok + c * CHUNK, CHUNK)],
                                  idx[b], sem_i[b]).wait()

        def store_start(c, b):
            pltpu.async_copy(rows[b], out_hbm.at[pl.ds(base_tok + c * CHUNK, CHUNK)],
                             sem_s[b])

        def store_wait(c, b):
            pltpu.make_async_copy(rows[b], out_hbm.at[pl.ds(base_tok + c * CHUNK, CHUNK)],
                                  sem_s[b]).wait()

        def compute(b, coff, lb):
            @plsc.parallel_loop(0, GPC, unroll=2)
            def group(g):
                gbase = g * LANES
                lg_raw = lb + gbase
                lbg = jnp.where(lg_raw >= L, lg_raw - L, lg_raw)
                mf16 = mf_v[pl.ds(coff + gbase, LANES)]
                for jj in range(LANES):
                    j = gbase + jj
                    sel = jnp.full((LANES,), jj, dtype=jnp.int32)
                    mf = mf16.at[sel].get(mode="promise_in_bounds")
                    l_raw = lbg + jj
                    l = jnp.where(l_raw >= L, l_raw - L, l_raw)
                    for kk in range(KD):
                        s = pl.ds(kk * LANES, LANES)
                        rows[b][j, s] = (rows[b][j, s] + pem_v[l, s]) * mf

            lbn = lb + CHUNK
            return jnp.where(lbn >= L, lbn - L, lbn)

        pltpu.sync_copy(x_hbm.at[pl.ds(base_tok, CHUNK)], idx0)
        gather_start(zero, 0)
        idx_start(one, 1)

        def step(s, lb):
            lbs = lb
            for cc in range(2):
                b, b1 = cc, 1 - cc
                c = 2 * s + cc
                gather_wait(c, b)
                @pl.when(s < n_steps - 1)
                def _():
                    idx_start(c + 2, b)
                @pl.when((c + 1 < n_chunks) & (c >= 1))
                def _():
                    store_wait(c - 1, b1)
                @pl.when(c + 1 < n_chunks)
                def _():
                    idx_wait(c + 1, b1)
                    gather_start(c + 1, b1)
                lbs = compute(b, c * CHUNK, lbs)
                store_start(c, b)
            return lbs

        lax.fori_loop(0, n_steps, step, zero)

        store_wait(n_chunks - 2, 0)
        store_wait(n_chunks - 1, 1)

    return k


def kernel(x, input_lengths, embedding_weight, pos_enc):
    B, L = x.shape
    V, D = embedding_weight.shape
    k = _build_sc_kernel(B, L, V, D)
    out = k(x.reshape(-1), input_lengths, embedding_weight, pos_enc)
    return out.reshape(B, L, D)


# speedup vs baseline: 1.4240x; 1.0201x over previous
import functools

import jax
import jax.numpy as jnp
from jax import lax
from jax.experimental import pallas as pl
from jax.experimental.pallas import tpu as pltpu
from jax.experimental.pallas import tpu_sc as plsc

LANES = 16
NBUF = 5
DEPTH = 3


def _build_sc_kernel(B, L, V, D):
    info = plsc.get_sparse_core_info()
    NC, NS = info.num_cores, info.num_subcores
    NW = NC * NS
    BL = B * L
    assert B % NW == 0
    rows_per_w = B // NW
    toks_per_w = rows_per_w * L
    CHUNK = 128
    assert toks_per_w % (NBUF * CHUNK) == 0
    n_chunks = toks_per_w // CHUNK
    n_steps = n_chunks // NBUF
    n_groups = toks_per_w // LANES
    assert D % LANES == 0
    KD = D // LANES
    GPC = CHUNK // LANES

    mesh = plsc.VectorSubcoreMesh(core_axis_name="c", subcore_axis_name="s")

    @functools.partial(
        pl.kernel,
        mesh=mesh,
        compiler_params=pltpu.CompilerParams(use_tc_tiling_on_sc=False),
        out_type=jax.ShapeDtypeStruct((BL, D), jnp.float32),
        scratch_types=[
            pltpu.VMEM((rows_per_w + LANES,), jnp.int32),
            pltpu.VMEM((L, D), jnp.float32),
            pltpu.VMEM((toks_per_w,), jnp.float32),
            pltpu.VMEM((toks_per_w,), jnp.int32),
            [pltpu.VMEM((CHUNK, D), jnp.float32) for _ in range(NBUF)],
            [pltpu.SemaphoreType.DMA for _ in range(NBUF)],
            [pltpu.SemaphoreType.DMA for _ in range(NBUF)],
        ],
    )
    def k(x_hbm, lens_hbm, emb_hbm, pe_hbm, out_hbm,
          lens_v, pem_v, mf_v, idx_all, rows, sem_g, sem_s):
        wid = lax.axis_index("s") * NC + lax.axis_index("c")
        base_row = wid * rows_per_w
        base_tok = wid * toks_per_w

        pltpu.sync_copy(lens_hbm.at[pl.ds(base_row, rows_per_w)],
                        lens_v.at[pl.ds(0, rows_per_w)])
        pltpu.sync_copy(pe_hbm, pem_v)
        pltpu.sync_copy(x_hbm.at[pl.ds(base_tok, toks_per_w)], idx_all)

        iota = lax.iota(jnp.int32, LANES)
        one = jnp.int32(1)
        zero = jnp.int32(0)

        def pe_scale(i, carry):
            for kk in range(KD):
                s = pl.ds(kk * LANES, LANES)
                pem_v[i, s] = pem_v[i, s] * jnp.float32(0.125)
            return carry

        lax.fori_loop(0, L, pe_scale, zero)

        def mask_group(g, carry):
            r0, l0 = carry
            l_raw = l0 + iota
            wrap = l_raw >= L
            l = jnp.where(wrap, l_raw - L, l_raw)
            rsel = jnp.where(wrap, one, zero)
            lens16 = lens_v[pl.ds(r0, LANES)]
            ln = lens16.at[rsel].get(mode="promise_in_bounds")
            mf = jnp.where(l < ln, jnp.float32(8.0), jnp.float32(0.0))
            mf_v[pl.ds(g * LANES, LANES)] = mf
            l0n = l0 + LANES
            over = l0n >= L
            l0n = jnp.where(over, l0n - L, l0n)
            r0n = jnp.where(over, r0 + one, r0)
            return (r0n, l0n)

        lax.fori_loop(0, n_groups, mask_group, (zero, zero))

        def gather_start(c, b):
            src = emb_hbm.at[idx_all.at[pl.ds(c * CHUNK, CHUNK)]]
            pltpu.async_copy(src, rows[b], sem_g[b])

        def gather_wait(c, b):
            src = emb_hbm.at[idx_all.at[pl.ds(c * CHUNK, CHUNK)]]
            pltpu.make_async_copy(src, rows[b], sem_g[b]).wait()

        def store_start(c, b):
            pltpu.async_copy(rows[b], out_hbm.at[pl.ds(base_tok + c * CHUNK, CHUNK)],
                             sem_s[b])

        def store_wait(c, b):
            pltpu.make_async_copy(rows[b], out_hbm.at[pl.ds(base_tok + c * CHUNK, CHUNK)],
                                  sem_s[b]).wait()

        def compute(b, coff, lb):
            @plsc.parallel_loop(0, GPC, unroll=2)
            def group(g):
                gbase = g * LANES
                lg_raw = lb + gbase
                lbg = jnp.where(lg_raw >= L, lg_raw - L, lg_raw)
                mf16 = mf_v[pl.ds(coff + gbase, LANES)]
                for jj in range(LANES):
                    j = gbase + jj
                    sel = jnp.full((LANES,), jj, dtype=jnp.int32)
                    mf = mf16.at[sel].get(mode="promise_in_bounds")
                    l_raw = lbg + jj
                    l = jnp.where(l_raw >= L, l_raw - L, l_raw)
                    for kk in range(KD):
                        s = pl.ds(kk * LANES, LANES)
                        rows[b][j, s] = (rows[b][j, s] + pem_v[l, s]) * mf

            lbn = lb + CHUNK
            return jnp.where(lbn >= L, lbn - L, lbn)

        for b in range(DEPTH):
            gather_start(jnp.int32(b), b)

        def step(s, lb):
            lbs = lb
            for cc in range(NBUF):
                b = cc
                c = NBUF * s + cc
                gather_wait(c, b)
                b3 = (cc + DEPTH) % NBUF
                @pl.when(c + DEPTH < n_chunks)
                def _():
                    @pl.when(c >= NBUF - DEPTH)
                    def _():
                        store_wait(c + DEPTH - NBUF, b3)
                    gather_start(c + DEPTH, b3)
                lbs = compute(b, c * CHUNK, lbs)
                store_start(c, b)
            return lbs

        lax.fori_loop(0, n_steps, step, zero)

        for i in range(NBUF):
            store_wait(n_chunks - NBUF + i, i)

    return k


def kernel(x, input_lengths, embedding_weight, pos_enc):
    B, L = x.shape
    V, D = embedding_weight.shape
    k = _build_sc_kernel(B, L, V, D)
    out = k(x.reshape(-1), input_lengths, embedding_weight, pos_enc)
    return out.reshape(B, L, D)
